# Initial kernel scaffold; baseline (speedup 1.0000x reference)
import jax
import jax.numpy as jnp
from jax.experimental import pallas as pl


def kernel(x, rank_emb):
    raise NotImplementedError("write your pallas kernel here")


# speedup vs baseline: 2.9159x; 2.9159x over previous
import jax
import jax.numpy as jnp
from jax.experimental import pallas as pl


_TB = 512


def _add_kernel(x_ref, r_ref, o_ref):
    o_ref[0] = x_ref[0] + r_ref[...]


def kernel(x, rank_emb):
    B, T, D = x.shape
    tb = _TB if T % _TB == 0 else T
    grid = (T // tb, B)
    return pl.pallas_call(
        _add_kernel,
        grid=grid,
        in_specs=[
            pl.BlockSpec((1, tb, D), lambda t, b: (b, t, 0)),
            pl.BlockSpec((tb, D), lambda t, b: (t, 0)),
        ],
        out_specs=pl.BlockSpec((1, tb, D), lambda t, b: (b, t, 0)),
        out_shape=jax.ShapeDtypeStruct((B, T, D), x.dtype),
    )(x, rank_emb[:T])


# speedup vs baseline: 3.3919x; 1.1633x over previous
import jax
import jax.numpy as jnp
from jax.experimental import pallas as pl


_TB = 1024


def _add_kernel(x_ref, r_ref, o_ref):
    o_ref[0] = x_ref[0] + r_ref[...]


def kernel(x, rank_emb):
    B, T, D = x.shape
    tb = _TB if T % _TB == 0 else T
    grid = (T // tb, B)
    return pl.pallas_call(
        _add_kernel,
        grid=grid,
        in_specs=[
            pl.BlockSpec((1, tb, D), lambda t, b: (b, t, 0)),
            pl.BlockSpec((tb, D), lambda t, b: (t, 0)),
        ],
        out_specs=pl.BlockSpec((1, tb, D), lambda t, b: (b, t, 0)),
        out_shape=jax.ShapeDtypeStruct((B, T, D), x.dtype),
    )(x, rank_emb[:T])


# speedup vs baseline: 3.6172x; 1.0664x over previous
import jax
import jax.numpy as jnp
from jax.experimental import pallas as pl


_TB = 2048


def _add_kernel(x_ref, r_ref, o_ref):
    o_ref[0] = x_ref[0] + r_ref[...]


def kernel(x, rank_emb):
    B, T, D = x.shape
    tb = _TB if T % _TB == 0 else T
    grid = (T // tb, B)
    return pl.pallas_call(
        _add_kernel,
        grid=grid,
        in_specs=[
            pl.BlockSpec((1, tb, D), lambda t, b: (b, t, 0)),
            pl.BlockSpec((tb, D), lambda t, b: (t, 0)),
        ],
        out_specs=pl.BlockSpec((1, tb, D), lambda t, b: (b, t, 0)),
        out_shape=jax.ShapeDtypeStruct((B, T, D), x.dtype),
    )(x, rank_emb[:T])


# speedup vs baseline: 3.6173x; 1.0000x over previous
import jax
import jax.numpy as jnp
from jax.experimental import pallas as pl
from jax.experimental.pallas import tpu as pltpu


_TB = 2048


def _add_kernel(x_ref, r_ref, o_ref):
    o_ref[0] = x_ref[0] + r_ref[...]


def kernel(x, rank_emb):
    B, T, D = x.shape
    tb = _TB if T % _TB == 0 else T
    grid = (T // tb, B)
    return pl.pallas_call(
        _add_kernel,
        grid=grid,
        in_specs=[
            pl.BlockSpec((1, tb, D), lambda t, b: (b, t, 0)),
            pl.BlockSpec((tb, D), lambda t, b: (t, 0)),
        ],
        out_specs=pl.BlockSpec((1, tb, D), lambda t, b: (b, t, 0)),
        out_shape=jax.ShapeDtypeStruct((B, T, D), x.dtype),
        compiler_params=pltpu.CompilerParams(
            dimension_semantics=("parallel", "parallel"),
        ),
    )(x, rank_emb[:T])


# speedup vs baseline: 3.6497x; 1.0089x over previous
import jax
import jax.numpy as jnp
from jax.experimental import pallas as pl
from jax.experimental.pallas import tpu as pltpu


_TB = 1024


def _add_kernel(x_ref, r_ref, o_ref):
    o_ref[...] = x_ref[...] + r_ref[None]


def kernel(x, rank_emb):
    B, T, D = x.shape
    tb = _TB if T % _TB == 0 else T
    bb = B if B <= 4 else 1
    grid = (T // tb, B // bb)
    return pl.pallas_call(
        _add_kernel,
        grid=grid,
        in_specs=[
            pl.BlockSpec((bb, tb, D), lambda t, b: (b, t, 0)),
            pl.BlockSpec((tb, D), lambda t, b: (t, 0)),
        ],
        out_specs=pl.BlockSpec((bb, tb, D), lambda t, b: (b, t, 0)),
        out_shape=jax.ShapeDtypeStruct((B, T, D), x.dtype),
        compiler_params=pltpu.CompilerParams(
            dimension_semantics=("parallel", "parallel"),
        ),
    )(x, rank_emb[:T])
